# Initial kernel scaffold; baseline (speedup 1.0000x reference)
import jax
import jax.numpy as jnp
from jax.experimental import pallas as pl


def kernel(feat_idx, w_weight, r_weight, r_noise_weight, fc_weight):
    raise NotImplementedError("write your pallas kernel here")


# speedup vs baseline: 776.2107x; 776.2107x over previous
import functools

import jax
import jax.numpy as jnp
from jax import lax
from jax.experimental import pallas as pl
from jax.experimental.pallas import tpu as pltpu
from jax.experimental.pallas import tpu_sc as plsc

ALPHA = 0.4
VOCAB_P1 = 100001
VPAD = 100352
B = 16384
LROW = 200
NC, NS, LANES = 2, 16, 16
NW = NC * NS
ROWS_PER_W = B // NW
CH = 16
NCHUNK = ROWS_PER_W // CH
UNROLL = 8


def _fuse_body(w_ref, r_ref, rn_ref, o_ref):
    o_ref[...] = (w_ref[...] + 0.4) * (
        (1.0 - ALPHA) * r_ref[...] + ALPHA * rn_ref[...])


def _fuse_table(w2, r2, rn2):
    return pl.pallas_call(
        _fuse_body,
        out_shape=jax.ShapeDtypeStruct((VPAD // 128, 128), jnp.float32),
    )(w2, r2, rn2)


def _sc_body(tbl_hbm, idx_hbm, c_hbm, out_hbm, tbl_v, idx_v, out_v, c_v):
    wid = lax.axis_index("s") * NC + lax.axis_index("c")
    pltpu.sync_copy(tbl_hbm, tbl_v)
    pltpu.sync_copy(c_hbm, c_v)
    cvec = c_v[...]
    c0 = cvec[0]
    c1 = cvec[1]
    row_iota = lax.iota(jnp.int32, LANES)
    base_row = wid * ROWS_PER_W

    row_off = row_iota * LROW

    def chunk_body(ch, _):
        r0 = base_row + ch * CH
        pltpu.sync_copy(idx_hbm.at[pl.ds(r0 * LROW, CH * LROW)], idx_v)

        def l_body(lb, accs):
            a0, a1, a2, a3 = accs
            lbase = lb * UNROLL
            for j in range(UNROLL):
                lvec = row_off + (lbase + j)
                idxv = plsc.load_gather(idx_v, [lvec])
                vals = plsc.load_gather(tbl_v, [idxv])
                if j % 4 == 0:
                    a0 = a0 + vals
                elif j % 4 == 1:
                    a1 = a1 + vals
                elif j % 4 == 2:
                    a2 = a2 + vals
                else:
                    a3 = a3 + vals
            return a0, a1, a2, a3

        z = jnp.zeros((LANES,), jnp.float32)
        a0, a1, a2, a3 = lax.fori_loop(0, LROW // UNROLL, l_body,
                                       (z, z, z, z))
        s = (a0 + a1) + (a2 + a3)
        plsc.store_scatter(out_v, [row_iota * 2], s * c0)
        plsc.store_scatter(out_v, [row_iota * 2 + 1], s * c1)
        pltpu.sync_copy(out_v, out_hbm.at[pl.ds(r0 * 2, CH * 2)])
        return _

    lax.fori_loop(0, NCHUNK, chunk_body, 0)


_sc_gather_sum = functools.partial(
    pl.kernel,
    out_type=jax.ShapeDtypeStruct((B * 2,), jnp.float32),
    mesh=plsc.VectorSubcoreMesh(core_axis_name="c", subcore_axis_name="s"),
    compiler_params=pltpu.CompilerParams(needs_layout_passes=False),
    scratch_types=[
        pltpu.VMEM((VPAD,), jnp.float32),
        pltpu.VMEM((CH * LROW,), jnp.int32),
        pltpu.VMEM((CH * 2,), jnp.float32),
        pltpu.VMEM((16,), jnp.float32),
    ],
)(_sc_body)


def kernel(feat_idx, w_weight, r_weight, r_noise_weight, fc_weight):
    pad = VPAD - VOCAB_P1
    w2 = jnp.pad(w_weight[:, 0], (0, pad)).reshape(VPAD // 128, 128)
    r2 = jnp.pad(r_weight[:, 0], (0, pad)).reshape(VPAD // 128, 128)
    rn2 = jnp.pad(r_noise_weight[:, 0], (0, pad)).reshape(VPAD // 128, 128)
    tbl = _fuse_table(w2, r2, rn2).reshape(VPAD)
    c = fc_weight[:, 0] - fc_weight[:, 1]
    c16 = jnp.pad(c, (0, 14))
    out_flat = _sc_gather_sum(tbl, feat_idx.reshape(B * LROW), c16)
    return out_flat.reshape(B, 2)


# speedup vs baseline: 803.1660x; 1.0347x over previous
import functools

import jax
import jax.numpy as jnp
from jax import lax
from jax.experimental import pallas as pl
from jax.experimental.pallas import tpu as pltpu
from jax.experimental.pallas import tpu_sc as plsc

ALPHA = 0.4
VOCAB_P1 = 100001
VPAD = 100352
B = 16384
LROW = 200
NC, NS, LANES = 2, 16, 16
NW = NC * NS
ROWS_PER_W = B // NW
CH = 32
NCHUNK = ROWS_PER_W // CH
NPAIR = NCHUNK // 2
UNROLL = 8


def _fuse_body(w_ref, r_ref, rn_ref, o_ref):
    o_ref[...] = (w_ref[...] + 0.4) * (
        (1.0 - ALPHA) * r_ref[...] + ALPHA * rn_ref[...])


def _fuse_table(w2, r2, rn2):
    return pl.pallas_call(
        _fuse_body,
        out_shape=jax.ShapeDtypeStruct((VPAD // 128, 128), jnp.float32),
    )(w2, r2, rn2)


def _sc_body(tbl_hbm, idx_hbm, c_hbm, out_hbm,
             tbl_v, idx_v0, idx_v1, out_v, c_v, sem_t, sem0, sem1):
    wid = lax.axis_index("s") * NC + lax.axis_index("c")
    base_row = wid * ROWS_PER_W
    row_iota = lax.iota(jnp.int32, LANES)

    def start_idx(ch, buf, sem):
        r0 = base_row + ch * CH
        pltpu.async_copy(idx_hbm.at[pl.ds(r0, CH)], buf, sem)

    def wait_idx(ch, buf, sem):
        r0 = base_row + ch * CH
        pltpu.make_async_copy(idx_hbm.at[pl.ds(r0, CH)], buf, sem).wait()

    h_tbl = pltpu.async_copy(tbl_hbm, tbl_v, sem_t)
    start_idx(0, idx_v0, sem0)
    start_idx(1, idx_v1, sem1)
    pltpu.sync_copy(c_hbm, c_v)
    cvec = c_v[...]
    c0 = cvec[0]
    c1 = cvec[1]
    h_tbl.wait()

    def process(ch, buf):
        for g in range(CH // LANES):
            rows = row_iota + g * LANES

            def l_body(lb, accs):
                a0, a1, a2, a3 = accs
                lbase = lb * UNROLL
                for j in range(UNROLL):
                    lvec = jnp.full((LANES,), lbase + j, jnp.int32)
                    idxv = plsc.load_gather(buf, [rows, lvec])
                    vals = plsc.load_gather(tbl_v, [idxv])
                    if j % 4 == 0:
                        a0 = a0 + vals
                    elif j % 4 == 1:
                        a1 = a1 + vals
                    elif j % 4 == 2:
                        a2 = a2 + vals
                    else:
                        a3 = a3 + vals
                return a0, a1, a2, a3

            z = jnp.zeros((LANES,), jnp.float32)
            a0, a1, a2, a3 = lax.fori_loop(0, LROW // UNROLL, l_body,
                                           (z, z, z, z))
            s = (a0 + a1) + (a2 + a3)
            opos = (ch * CH + g * LANES + row_iota) * 2
            plsc.store_scatter(out_v, [opos], s * c0)
            plsc.store_scatter(out_v, [opos + 1], s * c1)

    def pair_body(p, carry):
        c_a = p * 2

        wait_idx(c_a, idx_v0, sem0)
        process(c_a, idx_v0)

        @pl.when(p + 1 < NPAIR)
        def _():
            start_idx(c_a + 2, idx_v0, sem0)

        wait_idx(c_a + 1, idx_v1, sem1)
        process(c_a + 1, idx_v1)

        @pl.when(p + 1 < NPAIR)
        def _():
            start_idx(c_a + 3, idx_v1, sem1)

        return carry

    lax.fori_loop(0, NPAIR, pair_body, 0)
    pltpu.sync_copy(out_v, out_hbm.at[pl.ds(base_row * 2, ROWS_PER_W * 2)])


_sc_gather_sum = functools.partial(
    pl.kernel,
    out_type=jax.ShapeDtypeStruct((B * 2,), jnp.float32),
    mesh=plsc.VectorSubcoreMesh(core_axis_name="c", subcore_axis_name="s"),
    compiler_params=pltpu.CompilerParams(needs_layout_passes=False),
    scratch_types=[
        pltpu.VMEM((VPAD,), jnp.float32),
        pltpu.VMEM((CH, LROW), jnp.int32),
        pltpu.VMEM((CH, LROW), jnp.int32),
        pltpu.VMEM((ROWS_PER_W * 2,), jnp.float32),
        pltpu.VMEM((16,), jnp.float32),
        pltpu.SemaphoreType.DMA,
        pltpu.SemaphoreType.DMA,
        pltpu.SemaphoreType.DMA,
    ],
)(_sc_body)


def kernel(feat_idx, w_weight, r_weight, r_noise_weight, fc_weight):
    pad = VPAD - VOCAB_P1
    w2 = jnp.pad(w_weight[:, 0], (0, pad)).reshape(VPAD // 128, 128)
    r2 = jnp.pad(r_weight[:, 0], (0, pad)).reshape(VPAD // 128, 128)
    rn2 = jnp.pad(r_noise_weight[:, 0], (0, pad)).reshape(VPAD // 128, 128)
    tbl = _fuse_table(w2, r2, rn2).reshape(VPAD)
    c = fc_weight[:, 0] - fc_weight[:, 1]
    c16 = jnp.pad(c, (0, 14))
    out_flat = _sc_gather_sum(tbl, feat_idx, c16)
    return out_flat.reshape(B, 2)


# speedup vs baseline: 1089.7044x; 1.3568x over previous
import functools

import jax
import jax.numpy as jnp
from jax import lax
from jax.experimental import pallas as pl
from jax.experimental.pallas import tpu as pltpu
from jax.experimental.pallas import tpu_sc as plsc

ALPHA = 0.4
VOCAB_P1 = 100001
VPAD = 100352
B = 16384
LROW = 200
NC, NS, LANES = 2, 16, 16
NW = NC * NS
ROWS_PER_W = B // NW
CH = 32
NCHUNK = ROWS_PER_W // CH
NPAIR = NCHUNK // 2
UNROLL = 8


def _fuse_body(w_ref, r_ref, rn_ref, o_ref):
    o_ref[...] = (w_ref[...] + 0.4) * (
        (1.0 - ALPHA) * r_ref[...] + ALPHA * rn_ref[...])


def _fuse_table(w2, r2, rn2):
    return pl.pallas_call(
        _fuse_body,
        out_shape=jax.ShapeDtypeStruct((VPAD // 128, 128), jnp.float32),
    )(w2, r2, rn2)


def _sc_body(tbl_hbm, idx_hbm, c_hbm, out_hbm,
             tbl_v, idx_v0, idx_v1, out_v, c_v, sem_t, sem0, sem1):
    wid = lax.axis_index("s") * NC + lax.axis_index("c")
    base_row = wid * ROWS_PER_W
    row_iota = lax.iota(jnp.int32, LANES)

    def start_idx(ch, buf, sem):
        r0 = base_row + ch * CH
        pltpu.async_copy(idx_hbm.at[pl.ds(r0, CH)], buf, sem)

    def wait_idx(ch, buf, sem):
        r0 = base_row + ch * CH
        pltpu.make_async_copy(idx_hbm.at[pl.ds(r0, CH)], buf, sem).wait()

    h_tbl = pltpu.async_copy(tbl_hbm, tbl_v, sem_t)
    start_idx(0, idx_v0, sem0)
    start_idx(1, idx_v1, sem1)
    pltpu.sync_copy(c_hbm, c_v)
    cvec = c_v[...]
    c0 = cvec[0]
    c1 = cvec[1]
    h_tbl.wait()

    def process(ch, buf):
        for g in range(CH // LANES):
            rows = row_iota + g * LANES

            def l_body(lb, accs):
                a0, a1, a2, a3 = accs
                lbase = lb * UNROLL
                for j in range(UNROLL):
                    lv = row_iota + (lbase + j)
                    lv = jnp.where(lv >= LROW, lv - LROW, lv)
                    idxv = plsc.load_gather(buf, [rows, lv])
                    vals = plsc.load_gather(tbl_v, [idxv])
                    if j % 4 == 0:
                        a0 = a0 + vals
                    elif j % 4 == 1:
                        a1 = a1 + vals
                    elif j % 4 == 2:
                        a2 = a2 + vals
                    else:
                        a3 = a3 + vals
                return a0, a1, a2, a3

            z = jnp.zeros((LANES,), jnp.float32)
            a0, a1, a2, a3 = lax.fori_loop(0, LROW // UNROLL, l_body,
                                           (z, z, z, z))
            s = (a0 + a1) + (a2 + a3)
            opos = (ch * CH + g * LANES + row_iota) * 2
            plsc.store_scatter(out_v, [opos], s * c0)
            plsc.store_scatter(out_v, [opos + 1], s * c1)

    def pair_body(p, carry):
        c_a = p * 2

        wait_idx(c_a, idx_v0, sem0)
        process(c_a, idx_v0)

        @pl.when(p + 1 < NPAIR)
        def _():
            start_idx(c_a + 2, idx_v0, sem0)

        wait_idx(c_a + 1, idx_v1, sem1)
        process(c_a + 1, idx_v1)

        @pl.when(p + 1 < NPAIR)
        def _():
            start_idx(c_a + 3, idx_v1, sem1)

        return carry

    lax.fori_loop(0, NPAIR, pair_body, 0)
    pltpu.sync_copy(out_v, out_hbm.at[pl.ds(base_row * 2, ROWS_PER_W * 2)])


_sc_gather_sum = functools.partial(
    pl.kernel,
    out_type=jax.ShapeDtypeStruct((B * 2,), jnp.float32),
    mesh=plsc.VectorSubcoreMesh(core_axis_name="c", subcore_axis_name="s"),
    compiler_params=pltpu.CompilerParams(needs_layout_passes=False),
    scratch_types=[
        pltpu.VMEM((VPAD,), jnp.float32),
        pltpu.VMEM((CH, LROW), jnp.int32),
        pltpu.VMEM((CH, LROW), jnp.int32),
        pltpu.VMEM((ROWS_PER_W * 2,), jnp.float32),
        pltpu.VMEM((16,), jnp.float32),
        pltpu.SemaphoreType.DMA,
        pltpu.SemaphoreType.DMA,
        pltpu.SemaphoreType.DMA,
    ],
)(_sc_body)


def kernel(feat_idx, w_weight, r_weight, r_noise_weight, fc_weight):
    pad = VPAD - VOCAB_P1
    w2 = jnp.pad(w_weight[:, 0], (0, pad)).reshape(VPAD // 128, 128)
    r2 = jnp.pad(r_weight[:, 0], (0, pad)).reshape(VPAD // 128, 128)
    rn2 = jnp.pad(r_noise_weight[:, 0], (0, pad)).reshape(VPAD // 128, 128)
    tbl = _fuse_table(w2, r2, rn2).reshape(VPAD)
    c = fc_weight[:, 0] - fc_weight[:, 1]
    c16 = jnp.pad(c, (0, 14))
    out_flat = _sc_gather_sum(tbl, feat_idx, c16)
    return out_flat.reshape(B, 2)
